# baseline (speedup 1.0000x reference)
import math

import jax
import jax.numpy as jnp
from jax import lax
from jax.experimental import pallas as pl
from jax.experimental.pallas import tpu as pltpu
from jax.experimental.pallas import tpu_sc as plsc

D_MODEL = 64
SCALE = math.sqrt(D_MODEL)
NUM_WORKERS = 32
BLK = 128
SPU = 2
UNIT = SPU * BLK
LANES = 16
NBUF = 4
SBUF = 2


def _emb_body(xw_hbm, lut_hbm, out_hbm, idx_v, gbuf, sbuf, gsem, ssem):
    w = lax.axis_index("s") * 2 + lax.axis_index("c")
    nunits = xw_hbm.shape[1]
    ngroups = nunits // NBUF

    pltpu.sync_copy(xw_hbm.at[w], idx_v)

    iota = lax.iota(jnp.int32, LANES)
    iv_i = [(iota + d0) >> 3 for d0 in range(0, D_MODEL, LANES)]
    iv_k = iota & 7

    for b in range(NBUF):
        pltpu.async_copy(lut_hbm.at[idx_v.at[b]], gbuf.at[b], gsem.at[b])

    def group_body(g, carry):
        for b in range(NBUF):
            u = g * NBUF + b
            sb = b & 1
            pltpu.make_async_copy(
                lut_hbm.at[idx_v.at[u]], gbuf.at[b], gsem.at[b]).wait()

            if b >= SBUF:
                pltpu.make_async_copy(
                    sbuf.at[sb, :, :, :, pl.ds(0, BLK)],
                    out_hbm.at[pl.ds(0, SPU), :, w], ssem.at[sb]).wait()
            else:
                @pl.when(g >= 1)
                def _wait_store():
                    pltpu.make_async_copy(
                        sbuf.at[sb, :, :, :, pl.ds(0, BLK)],
                        out_hbm.at[pl.ds(0, SPU), :, w], ssem.at[sb]).wait()

            for h in range(SPU):
                def tok_body(t, c2, h=h):
                    tv = jnp.full((LANES,), t, jnp.int32)
                    for c in range(D_MODEL // LANES):
                        v = gbuf[b, h * BLK + t, pl.ds(c * LANES, LANES)]
                        plsc.store_scatter(
                            sbuf.at[sb, h], [iv_i[c], iv_k, tv], v * SCALE)
                    return c2

                lax.fori_loop(0, BLK, tok_body, 0, unroll=4)

            pltpu.async_copy(
                sbuf.at[sb, :, :, :, pl.ds(0, BLK)],
                out_hbm.at[pl.ds(SPU * u, SPU), :, w], ssem.at[sb])

            @pl.when(g < ngroups - 1)
            def _prefetch():
                pltpu.async_copy(
                    lut_hbm.at[idx_v.at[u + NBUF]], gbuf.at[b], gsem.at[b])
        return carry

    lax.fori_loop(0, ngroups, group_body, 0)

    for sb in range(SBUF):
        pltpu.make_async_copy(
            sbuf.at[sb, :, :, :, pl.ds(0, BLK)],
            out_hbm.at[pl.ds(0, SPU), :, w], ssem.at[sb]).wait()


def kernel(x, lut):
    bsz, seq = x.shape
    nblocks = bsz // BLK
    assert nblocks == NUM_WORKERS and seq % (SPU * NBUF) == 0
    nunits = seq // SPU
    xw = x.T.reshape(seq, nblocks, BLK).transpose(1, 0, 2)
    xw = xw.reshape(nblocks, nunits, UNIT)
    mesh = plsc.VectorSubcoreMesh(core_axis_name="c", subcore_axis_name="s")
    out = pl.kernel(
        _emb_body,
        out_type=jax.ShapeDtypeStruct(
            (seq, 8, nblocks, 8, BLK), jnp.float32),
        mesh=mesh,
        scratch_types=[
            pltpu.VMEM((nunits, UNIT), jnp.int32),
            pltpu.VMEM((NBUF, UNIT, D_MODEL), jnp.float32),
            pltpu.VMEM((SBUF, SPU, 8, 8, BLK + 1), jnp.float32),
            pltpu.SemaphoreType.DMA((NBUF,)),
            pltpu.SemaphoreType.DMA((SBUF,)),
        ],
        compiler_params=pltpu.CompilerParams(
            use_tc_tiling_on_sc=False, needs_layout_passes=False),
    )(xw, lut)
    out = out.transpose(2, 4, 0, 1, 3).reshape(bsz, seq, D_MODEL)
    return out


# speedup vs baseline: 1.0189x; 1.0189x over previous
import math

import jax
import jax.numpy as jnp
from jax import lax
from jax.experimental import pallas as pl
from jax.experimental.pallas import tpu as pltpu
from jax.experimental.pallas import tpu_sc as plsc

D_MODEL = 64
SCALE = math.sqrt(D_MODEL)
NUM_WORKERS = 32
CHUNK = 128
LANES = 16
NBUF = 4


def _emb_body(xt_hbm, lut_hbm, out_hbm, idx_v, gbuf, sbuf, gsem, ssem):
    w = lax.axis_index("s") * 2 + lax.axis_index("c")
    nunits = xt_hbm.shape[0]
    ngroups = nunits // NBUF

    pltpu.sync_copy(xt_hbm.at[:, pl.ds(w * CHUNK, CHUNK)], idx_v)

    iota = lax.iota(jnp.int32, LANES)
    iv_i = [(iota + d0) >> 3 for d0 in range(0, D_MODEL, LANES)]
    iv_k = iota & 7

    for b in range(NBUF):
        pltpu.async_copy(lut_hbm.at[idx_v.at[b]], gbuf.at[b], gsem.at[b])

    def group_body(g, carry):
        for b in range(NBUF):
            u = g * NBUF + b
            pltpu.make_async_copy(
                lut_hbm.at[idx_v.at[u]], gbuf.at[b], gsem.at[b]).wait()

            @pl.when(g >= 1)
            def _wait_store():
                pltpu.make_async_copy(
                    sbuf.at[b, :, :, pl.ds(0, CHUNK)],
                    out_hbm.at[0, :, w], ssem.at[b]).wait()

            def tok_body(t, c2):
                tv = jnp.full((LANES,), t, jnp.int32)
                for c in range(D_MODEL // LANES):
                    v = gbuf[b, t, pl.ds(c * LANES, LANES)] * SCALE
                    plsc.store_scatter(sbuf.at[b], [iv_i[c], iv_k, tv], v)
                return c2

            lax.fori_loop(0, CHUNK, tok_body, 0, unroll=4)

            pltpu.async_copy(
                sbuf.at[b, :, :, pl.ds(0, CHUNK)], out_hbm.at[u, :, w],
                ssem.at[b])

            @pl.when(g < ngroups - 1)
            def _prefetch():
                pltpu.async_copy(
                    lut_hbm.at[idx_v.at[u + NBUF]], gbuf.at[b], gsem.at[b])
        return carry

    lax.fori_loop(0, ngroups, group_body, 0)

    for b in range(NBUF):
        pltpu.make_async_copy(
            sbuf.at[b, :, :, pl.ds(0, CHUNK)], out_hbm.at[0, :, w],
            ssem.at[b]).wait()


def kernel(x, lut):
    bsz, seq = x.shape
    nblocks = bsz // CHUNK
    assert nblocks == NUM_WORKERS and seq % NBUF == 0
    xt = x.T
    mesh = plsc.VectorSubcoreMesh(core_axis_name="c", subcore_axis_name="s")
    out = pl.kernel(
        _emb_body,
        out_type=jax.ShapeDtypeStruct(
            (seq, 8, nblocks, 8, CHUNK), jnp.float32),
        mesh=mesh,
        scratch_types=[
            pltpu.VMEM((seq, CHUNK), jnp.int32),
            pltpu.VMEM((NBUF, CHUNK, D_MODEL), jnp.float32),
            pltpu.VMEM((NBUF, 8, 8, CHUNK + 1), jnp.float32),
            pltpu.SemaphoreType.DMA((NBUF,)),
            pltpu.SemaphoreType.DMA((NBUF,)),
        ],
        compiler_params=pltpu.CompilerParams(
            use_tc_tiling_on_sc=False, needs_layout_passes=False),
    )(xt, lut)
    out = out.transpose(2, 4, 0, 1, 3).reshape(bsz, seq, D_MODEL)
    return out
